# baseline (speedup 1.0000x reference)
import functools
import math

import jax
import jax.numpy as jnp
from jax import lax
from jax.experimental import pallas as pl
from jax.experimental.pallas import tpu as pltpu
from jax.experimental.pallas import tpu_sc as plsc

B, L, D, F, E = 1, 2048, 1024, 4096, 64
T = B * L
TOP_K = 1
CAP_FACTOR = 1.25
C = max(1, int(math.ceil(CAP_FACTOR * (T * TOP_K / E))))
C_PAD = 48
S = E * C_PAD
FT = 1024
NF = F // FT

NUM_SC = 2
NUM_SUBCORES = 16
NW = NUM_SC * NUM_SUBCORES


def _router_body(x_ref, gw_ref, gb_ref, gmat_ref, pmat_ref, oslot_ref):
    x = x_ref[...]
    logits = lax.dot_general(x, gw_ref[...], (((1,), (1,)), ((), ())),
                             preferred_element_type=jnp.float32)
    logits = logits + gb_ref[...]
    m = jnp.max(logits, axis=1, keepdims=True)
    ssum = jnp.sum(jnp.exp(logits - m), axis=1, keepdims=True)
    prb = 1.0 / ssum
    iota_e = lax.broadcasted_iota(jnp.int32, (T, E), 1)
    idx = jnp.min(jnp.where(logits == m, iota_e, E), axis=1, keepdims=True)
    oh_e = (iota_e == idx).astype(jnp.float32)

    CH = 256
    tri = (lax.broadcasted_iota(jnp.int32, (CH, CH), 0)
           >= lax.broadcasted_iota(jnp.int32, (CH, CH), 1)).astype(jnp.float32)
    base = jnp.zeros((1, E), jnp.float32)
    chunks = []
    for k in range(T // CH):
        ohk = oh_e[k * CH:(k + 1) * CH, :]
        chunks.append(lax.dot_general(tri, ohk, (((1,), (0,)), ((), ())),
                                      preferred_element_type=jnp.float32) + base)
        base = base + jnp.sum(ohk, axis=0, keepdims=True)
    cnt = jnp.concatenate(chunks, axis=0)
    pos = jnp.sum(cnt * oh_e, axis=1, keepdims=True) - 1.0
    keep = pos < float(C)
    pos_i = pos.astype(jnp.int32)
    oslot_ref[...] = idx * C_PAD + jnp.where(keep, pos_i, C_PAD - 1)

    iota_c = lax.broadcasted_iota(jnp.int32, (T, C_PAD), 1)
    oh_c = ((iota_c == pos_i) & keep).astype(jnp.float32)
    tvec = lax.broadcasted_iota(jnp.int32, (T, 1), 0).astype(jnp.float32)
    gmat_ref[...] = lax.dot_general(oh_e * tvec, oh_c, (((0,), (0,)), ((), ())),
                                    preferred_element_type=jnp.float32)
    pmat_ref[...] = lax.dot_general(oh_e * prb, oh_c, (((0,), (0,)), ((), ())),
                                    preferred_element_type=jnp.float32)


def _router(x, gw, gb2):
    return pl.pallas_call(
        _router_body,
        out_shape=[
            jax.ShapeDtypeStruct((E, C_PAD), jnp.float32),
            jax.ShapeDtypeStruct((E, C_PAD), jnp.float32),
            jax.ShapeDtypeStruct((T, 1), jnp.int32),
        ],
    )(x, gw, gb2)


@functools.cache
def _make_sc_gather(n_rows, d):
    rpw = n_rows // NW
    assert n_rows % (8 * NW) == 0
    mesh = plsc.VectorSubcoreMesh(core_axis_name="c", subcore_axis_name="s",
                                  num_cores=NUM_SC, num_subcores=NUM_SUBCORES)

    @functools.partial(
        pl.kernel,
        out_type=jax.ShapeDtypeStruct((n_rows, d), jnp.float32),
        mesh=mesh,
        scratch_types=[
            pltpu.VMEM((rpw,), jnp.int32),
            pltpu.VMEM((rpw, d), jnp.float32),
            pltpu.SemaphoreType.DMA,
        ],
    )
    def gather(table_hbm, idx_hbm, out_hbm, idx_v, rows_v, sem):
        wid = lax.axis_index("s") * NUM_SC + lax.axis_index("c")
        base = wid * rpw
        pltpu.sync_copy(idx_hbm.at[pl.ds(base, rpw)], idx_v)
        pltpu.async_copy(table_hbm.at[idx_v], rows_v, sem).wait()
        pltpu.sync_copy(rows_v, out_hbm.at[pl.ds(base, rpw)])

    return gather


def _sc_gather(table, idx):
    return _make_sc_gather(idx.shape[0], table.shape[1])(table, idx)


def _ffn_body(xg_ref, w1_ref, b1_ref, w2_ref, b2_ref, pm_ref, h2_ref, acc_ref):
    f = pl.program_id(1)
    xv = xg_ref[0]
    h1 = lax.dot_general(xv, w1_ref[0], (((1,), (1,)), ((), ())),
                         preferred_element_type=jnp.float32)
    h1 = jnp.maximum(h1 + b1_ref[0], 0.0)
    part = lax.dot_general(h1, w2_ref[0], (((1,), (1,)), ((), ())),
                           preferred_element_type=jnp.float32)

    @pl.when(f == 0)
    def _():
        acc_ref[...] = part

    @pl.when(f > 0)
    def _():
        acc_ref[...] = acc_ref[...] + part

    @pl.when(f == NF - 1)
    def _():
        h2_ref[0] = (acc_ref[...] + b2_ref[0]) * pm_ref[0]


def _ffn(xg3, w1, b1r, w2, b2r, pm3):
    return pl.pallas_call(
        _ffn_body,
        grid=(E, NF),
        in_specs=[
            pl.BlockSpec((1, C_PAD, D), lambda e, f: (e, 0, 0)),
            pl.BlockSpec((1, FT, D), lambda e, f: (e, f, 0)),
            pl.BlockSpec((1, 1, FT), lambda e, f: (e, 0, f)),
            pl.BlockSpec((1, D, FT), lambda e, f: (e, 0, f)),
            pl.BlockSpec((1, 1, D), lambda e, f: (e, 0, 0)),
            pl.BlockSpec((1, C_PAD, 1), lambda e, f: (e, 0, 0)),
        ],
        out_specs=pl.BlockSpec((1, C_PAD, D), lambda e, f: (e, 0, 0)),
        out_shape=jax.ShapeDtypeStruct((E, C_PAD, D), jnp.float32),
        scratch_shapes=[pltpu.VMEM((C_PAD, D), jnp.float32)],
        compiler_params=pltpu.CompilerParams(
            dimension_semantics=("arbitrary", "arbitrary")),
    )(xg3, w1, b1r, w2, b2r, pm3)


def kernel(h, gate_w, gate_b, w1, b1, w2, b2):
    x = h.reshape(T, D)
    gmat, pmat, oslot = _router(x, gate_w, gate_b.reshape(1, E))
    gidx = gmat.reshape(S).astype(jnp.int32)
    xg = _sc_gather(x, gidx)
    h2 = _ffn(xg.reshape(E, C_PAD, D), w1, b1.reshape(E, 1, F), w2,
              b2.reshape(E, 1, D), pmat.reshape(E, C_PAD, 1))
    out = _sc_gather(h2.reshape(S, D), oslot.reshape(T))
    return out.reshape(B, L, D)


# speedup vs baseline: 1.1078x; 1.1078x over previous
import functools
import math

import jax
import jax.numpy as jnp
from jax import lax
from jax.experimental import pallas as pl
from jax.experimental.pallas import tpu as pltpu
from jax.experimental.pallas import tpu_sc as plsc

B, L, D, F, E = 1, 2048, 1024, 4096, 64
T = B * L
TOP_K = 1
CAP_FACTOR = 1.25
C = max(1, int(math.ceil(CAP_FACTOR * (T * TOP_K / E))))
C_PAD = 48
S = E * C_PAD
FT = 2048
NF = F // FT

NUM_SC = 2
NUM_SUBCORES = 16
NW = NUM_SC * NUM_SUBCORES


def _router_body(x_ref, gw_ref, gb_ref, gmat_ref, pmat_ref, oslot_ref):
    x = x_ref[...]
    logits = lax.dot_general(x, gw_ref[...], (((1,), (1,)), ((), ())),
                             preferred_element_type=jnp.float32)
    logits = logits + gb_ref[...]
    m = jnp.max(logits, axis=1, keepdims=True)
    ssum = jnp.sum(jnp.exp(logits - m), axis=1, keepdims=True)
    prb = 1.0 / ssum
    iota_e = lax.broadcasted_iota(jnp.int32, (T, E), 1)
    idx = jnp.min(jnp.where(logits == m, iota_e, E), axis=1, keepdims=True)
    oh_e = (iota_e == idx).astype(jnp.float32)

    CH = 256
    tri = (lax.broadcasted_iota(jnp.int32, (CH, CH), 0)
           >= lax.broadcasted_iota(jnp.int32, (CH, CH), 1)).astype(jnp.float32)
    base = jnp.zeros((1, E), jnp.float32)
    chunks = []
    for k in range(T // CH):
        ohk = oh_e[k * CH:(k + 1) * CH, :]
        chunks.append(lax.dot_general(tri, ohk, (((1,), (0,)), ((), ())),
                                      preferred_element_type=jnp.float32) + base)
        base = base + jnp.sum(ohk, axis=0, keepdims=True)
    cnt = jnp.concatenate(chunks, axis=0)
    pos = jnp.sum(cnt * oh_e, axis=1, keepdims=True) - 1.0
    keep = pos < float(C)
    pos_i = pos.astype(jnp.int32)
    oslot_ref[...] = idx * C_PAD + jnp.where(keep, pos_i, C_PAD - 1)

    iota_c = lax.broadcasted_iota(jnp.int32, (T, C_PAD), 1)
    oh_c = ((iota_c == pos_i) & keep).astype(jnp.float32)
    ti = lax.broadcasted_iota(jnp.int32, (T, 1), 0)
    hi_f = lax.shift_right_logical(ti, 8).astype(jnp.float32)
    lo_f = jnp.bitwise_and(ti, 255).astype(jnp.float32)
    g_hi = lax.dot_general(oh_e * hi_f, oh_c, (((0,), (0,)), ((), ())),
                           preferred_element_type=jnp.float32)
    g_lo = lax.dot_general(oh_e * lo_f, oh_c, (((0,), (0,)), ((), ())),
                           preferred_element_type=jnp.float32)
    pmat = lax.dot_general(oh_e * prb, oh_c, (((0,), (0,)), ((), ())),
                           preferred_element_type=jnp.float32)
    slot_iota = (lax.broadcasted_iota(jnp.int32, (E, C_PAD), 0) * C_PAD
                 + lax.broadcasted_iota(jnp.int32, (E, C_PAD), 1))
    fallback = jnp.bitwise_and(slot_iota, T - 1).astype(jnp.float32)
    gmat_ref[...] = jnp.where(pmat > 0.0, g_hi * 256.0 + g_lo, fallback)
    pmat_ref[...] = pmat


def _router(x, gw, gb2):
    return pl.pallas_call(
        _router_body,
        out_shape=[
            jax.ShapeDtypeStruct((E, C_PAD), jnp.float32),
            jax.ShapeDtypeStruct((E, C_PAD), jnp.float32),
            jax.ShapeDtypeStruct((T, 1), jnp.int32),
        ],
    )(x, gw, gb2)


@functools.cache
def _make_sc_gather(n_rows, d):
    rpw = n_rows // NW
    assert n_rows % (8 * NW) == 0
    mesh = plsc.VectorSubcoreMesh(core_axis_name="c", subcore_axis_name="s",
                                  num_cores=NUM_SC, num_subcores=NUM_SUBCORES)

    @functools.partial(
        pl.kernel,
        out_type=jax.ShapeDtypeStruct((n_rows, d), jnp.float32),
        mesh=mesh,
        scratch_types=[
            pltpu.VMEM((rpw,), jnp.int32),
            pltpu.VMEM((rpw, d), jnp.float32),
            pltpu.SemaphoreType.DMA,
        ],
    )
    def gather(table_hbm, idx_hbm, out_hbm, idx_v, rows_v, sem):
        wid = lax.axis_index("s") * NUM_SC + lax.axis_index("c")
        base = wid * rpw
        pltpu.sync_copy(idx_hbm.at[pl.ds(base, rpw)], idx_v)
        pltpu.async_copy(table_hbm.at[idx_v], rows_v, sem).wait()
        pltpu.sync_copy(rows_v, out_hbm.at[pl.ds(base, rpw)])

    return gather


def _sc_gather(table, idx):
    return _make_sc_gather(idx.shape[0], table.shape[1])(table, idx)


def _ffn_body(xg_ref, w1_ref, b1_ref, w2_ref, b2_ref, pm_ref, h2_ref, acc_ref):
    f = pl.program_id(1)
    xv = xg_ref[0]
    h1 = lax.dot_general(xv, w1_ref[0], (((1,), (1,)), ((), ())),
                         preferred_element_type=jnp.float32)
    h1 = jnp.maximum(h1 + b1_ref[0], 0.0)
    part = lax.dot_general(h1, w2_ref[0], (((1,), (1,)), ((), ())),
                           preferred_element_type=jnp.float32)

    @pl.when(f == 0)
    def _():
        acc_ref[...] = part

    @pl.when(f > 0)
    def _():
        acc_ref[...] = acc_ref[...] + part

    @pl.when(f == NF - 1)
    def _():
        h2_ref[0] = (acc_ref[...] + b2_ref[0]) * pm_ref[0]


def _ffn(xg3, w1, b1r, w2, b2r, pm3):
    return pl.pallas_call(
        _ffn_body,
        grid=(E, NF),
        in_specs=[
            pl.BlockSpec((1, C_PAD, D), lambda e, f: (e, 0, 0)),
            pl.BlockSpec((1, FT, D), lambda e, f: (e, f, 0)),
            pl.BlockSpec((1, 1, FT), lambda e, f: (e, 0, f)),
            pl.BlockSpec((1, D, FT), lambda e, f: (e, 0, f)),
            pl.BlockSpec((1, 1, D), lambda e, f: (e, 0, 0)),
            pl.BlockSpec((1, C_PAD, 1), lambda e, f: (e, 0, 0)),
        ],
        out_specs=pl.BlockSpec((1, C_PAD, D), lambda e, f: (e, 0, 0)),
        out_shape=jax.ShapeDtypeStruct((E, C_PAD, D), jnp.float32),
        scratch_shapes=[pltpu.VMEM((C_PAD, D), jnp.float32)],
        compiler_params=pltpu.CompilerParams(
            dimension_semantics=("arbitrary", "arbitrary")),
    )(xg3, w1, b1r, w2, b2r, pm3)


def kernel(h, gate_w, gate_b, w1, b1, w2, b2):
    x = h.reshape(T, D)
    gmat, pmat, oslot = _router(x, gate_w, gate_b.reshape(1, E))
    gidx = gmat.reshape(S).astype(jnp.int32)
    xg = _sc_gather(x, gidx)
    h2 = _ffn(xg.reshape(E, C_PAD, D), w1, b1.reshape(E, 1, F), w2,
              b2.reshape(E, 1, D), pmat.reshape(E, C_PAD, 1))
    out = _sc_gather(h2.reshape(S, D), oslot.reshape(T))
    return out.reshape(B, L, D)


# speedup vs baseline: 1.1220x; 1.0128x over previous
import functools
import math

import jax
import jax.numpy as jnp
from jax import lax
from jax.experimental import pallas as pl
from jax.experimental.pallas import tpu as pltpu
from jax.experimental.pallas import tpu_sc as plsc

B, L, D, F, E = 1, 2048, 1024, 4096, 64
T = B * L
TOP_K = 1
CAP_FACTOR = 1.25
C = max(1, int(math.ceil(CAP_FACTOR * (T * TOP_K / E))))
C_PAD = 48
S = E * C_PAD
FT = 2048
NF = F // FT

NUM_SC = 2
NUM_SUBCORES = 16
NW = NUM_SC * NUM_SUBCORES


def _router_body(x_ref, gw_ref, gb_ref, gmat_ref, pmat_ref, s2t_ref):
    x = x_ref[...]
    logits = lax.dot_general(x, gw_ref[...], (((1,), (1,)), ((), ())),
                             preferred_element_type=jnp.float32)
    logits = logits + gb_ref[...]
    m = jnp.max(logits, axis=1, keepdims=True)
    ssum = jnp.sum(jnp.exp(logits - m), axis=1, keepdims=True)
    prb = 1.0 / ssum
    iota_e = lax.broadcasted_iota(jnp.int32, (T, E), 1)
    idx = jnp.min(jnp.where(logits == m, iota_e, E), axis=1, keepdims=True)
    oh_e = (iota_e == idx).astype(jnp.float32)

    CH = 256
    tri = (lax.broadcasted_iota(jnp.int32, (CH, CH), 0)
           >= lax.broadcasted_iota(jnp.int32, (CH, CH), 1)).astype(jnp.float32)
    base = jnp.zeros((1, E), jnp.float32)
    chunks = []
    for k in range(T // CH):
        ohk = oh_e[k * CH:(k + 1) * CH, :]
        chunks.append(lax.dot_general(tri, ohk, (((1,), (0,)), ((), ())),
                                      preferred_element_type=jnp.float32) + base)
        base = base + jnp.sum(ohk, axis=0, keepdims=True)
    cnt = jnp.concatenate(chunks, axis=0)
    pos = jnp.sum(cnt * oh_e, axis=1, keepdims=True) - 1.0
    keep = pos < float(C)
    pos_i = pos.astype(jnp.int32)

    iota_c = lax.broadcasted_iota(jnp.int32, (T, C_PAD), 1)
    oh_c = ((iota_c == pos_i) & keep).astype(jnp.float32)
    ti = lax.broadcasted_iota(jnp.int32, (T, 1), 0)
    hi_f = lax.shift_right_logical(ti, 8).astype(jnp.float32)
    lo_f = jnp.bitwise_and(ti, 255).astype(jnp.float32)
    g_hi = lax.dot_general(oh_e * hi_f, oh_c, (((0,), (0,)), ((), ())),
                           preferred_element_type=jnp.float32)
    g_lo = lax.dot_general(oh_e * lo_f, oh_c, (((0,), (0,)), ((), ())),
                           preferred_element_type=jnp.float32)
    pmat = lax.dot_general(oh_e * prb, oh_c, (((0,), (0,)), ((), ())),
                           preferred_element_type=jnp.float32)
    slot_iota = (lax.broadcasted_iota(jnp.int32, (E, C_PAD), 0) * C_PAD
                 + lax.broadcasted_iota(jnp.int32, (E, C_PAD), 1))
    fallback = jnp.bitwise_and(slot_iota, T - 1).astype(jnp.float32)
    g = g_hi * 256.0 + g_lo
    gmat_ref[...] = jnp.where(pmat > 0.0, g, fallback)
    pmat_ref[...] = pmat
    s2t_ref[...] = jnp.where(pmat > 0.0, g, -1.0)


def _router(x, gw, gb2):
    return pl.pallas_call(
        _router_body,
        out_shape=[
            jax.ShapeDtypeStruct((E, C_PAD), jnp.float32),
            jax.ShapeDtypeStruct((E, C_PAD), jnp.float32),
            jax.ShapeDtypeStruct((E, C_PAD), jnp.float32),
        ],
    )(x, gw, gb2)


@functools.cache
def _make_sc_gather(n_rows, d):
    rpw = n_rows // NW
    assert n_rows % (8 * NW) == 0
    mesh = plsc.VectorSubcoreMesh(core_axis_name="c", subcore_axis_name="s",
                                  num_cores=NUM_SC, num_subcores=NUM_SUBCORES)

    @functools.partial(
        pl.kernel,
        out_type=jax.ShapeDtypeStruct((n_rows, d), jnp.float32),
        mesh=mesh,
        scratch_types=[
            pltpu.VMEM((rpw,), jnp.int32),
            pltpu.VMEM((rpw, d), jnp.float32),
            pltpu.SemaphoreType.DMA,
        ],
    )
    def gather(table_hbm, idx_hbm, out_hbm, idx_v, rows_v, sem):
        wid = lax.axis_index("s") * NUM_SC + lax.axis_index("c")
        base = wid * rpw
        pltpu.sync_copy(idx_hbm.at[pl.ds(base, rpw)], idx_v)
        pltpu.async_copy(table_hbm.at[idx_v], rows_v, sem).wait()
        pltpu.sync_copy(rows_v, out_hbm.at[pl.ds(base, rpw)])

    return gather


def _sc_gather(table, idx):
    return _make_sc_gather(idx.shape[0], table.shape[1])(table, idx)


def _ffn_body(s2t_ref, xg_ref, w1_ref, b1_ref, w2_ref, b2_ref, pm_ref,
              out_ref, acc_ref):
    e = pl.program_id(0)
    f = pl.program_id(1)

    @pl.when((e == 0) & (f == 0))
    def _():
        out_ref[...] = jnp.zeros((T, D), jnp.float32)

    xv = xg_ref[0]
    h1 = lax.dot_general(xv, w1_ref[0], (((1,), (1,)), ((), ())),
                         preferred_element_type=jnp.float32)
    h1 = jnp.maximum(h1 + b1_ref[0], 0.0)
    part = lax.dot_general(h1, w2_ref[0], (((1,), (1,)), ((), ())),
                           preferred_element_type=jnp.float32)

    @pl.when(f == 0)
    def _():
        acc_ref[...] = part

    @pl.when(f > 0)
    def _():
        acc_ref[...] = acc_ref[...] + part

    @pl.when(f == NF - 1)
    def _():
        res = (acc_ref[...] + b2_ref[0]) * pm_ref[0]
        for c in range(C):
            tok = s2t_ref[e * C_PAD + c]

            @pl.when(tok >= 0)
            def _():
                out_ref[pl.ds(tok, 1), :] = res[c:c + 1, :]


def _ffn(s2t, xg3, w1, b1r, w2, b2r, pm3):
    grid_spec = pltpu.PrefetchScalarGridSpec(
        num_scalar_prefetch=1,
        grid=(E, NF),
        in_specs=[
            pl.BlockSpec((1, C_PAD, D), lambda e, f, s: (e, 0, 0)),
            pl.BlockSpec((1, FT, D), lambda e, f, s: (e, f, 0)),
            pl.BlockSpec((1, 1, FT), lambda e, f, s: (e, 0, f)),
            pl.BlockSpec((1, D, FT), lambda e, f, s: (e, 0, f)),
            pl.BlockSpec((1, 1, D), lambda e, f, s: (e, 0, 0)),
            pl.BlockSpec((1, C_PAD, 1), lambda e, f, s: (e, 0, 0)),
        ],
        out_specs=pl.BlockSpec((T, D), lambda e, f, s: (0, 0)),
        scratch_shapes=[pltpu.VMEM((C_PAD, D), jnp.float32)],
    )
    return pl.pallas_call(
        _ffn_body,
        grid_spec=grid_spec,
        out_shape=jax.ShapeDtypeStruct((T, D), jnp.float32),
        compiler_params=pltpu.CompilerParams(
            dimension_semantics=("arbitrary", "arbitrary")),
    )(s2t, xg3, w1, b1r, w2, b2r, pm3)


def kernel(h, gate_w, gate_b, w1, b1, w2, b2):
    x = h.reshape(T, D)
    gmat, pmat, s2t = _router(x, gate_w, gate_b.reshape(1, E))
    gidx = gmat.reshape(S).astype(jnp.int32)
    xg = _sc_gather(x, gidx)
    out = _ffn(s2t.reshape(S).astype(jnp.int32), xg.reshape(E, C_PAD, D),
               w1, b1.reshape(E, 1, F), w2, b2.reshape(E, 1, D),
               pmat.reshape(E, C_PAD, 1))
    return out.reshape(B, L, D)


# speedup vs baseline: 1.1259x; 1.0035x over previous
import functools
import math

import jax
import jax.numpy as jnp
from jax import lax
from jax.experimental import pallas as pl
from jax.experimental.pallas import tpu as pltpu
from jax.experimental.pallas import tpu_sc as plsc

B, L, D, F, E = 1, 2048, 1024, 4096, 64
T = B * L
TOP_K = 1
CAP_FACTOR = 1.25
C = max(1, int(math.ceil(CAP_FACTOR * (T * TOP_K / E))))
C_PAD = 40
S = E * C_PAD
FT = 2048
NF = F // FT

NUM_SC = 2
NUM_SUBCORES = 16
NW = NUM_SC * NUM_SUBCORES


def _router_body(x_ref, gw_ref, gb_ref, gmat_ref, pmat_ref, s2t_ref):
    x = x_ref[...]
    logits = lax.dot_general(x, gw_ref[...], (((1,), (1,)), ((), ())),
                             preferred_element_type=jnp.float32)
    logits = logits + gb_ref[...]
    m = jnp.max(logits, axis=1, keepdims=True)
    ssum = jnp.sum(jnp.exp(logits - m), axis=1, keepdims=True)
    prb = 1.0 / ssum
    iota_e = lax.broadcasted_iota(jnp.int32, (T, E), 1)
    idx = jnp.min(jnp.where(logits == m, iota_e, E), axis=1, keepdims=True)
    oh_e = (iota_e == idx).astype(jnp.float32)

    CH = 256
    tri = (lax.broadcasted_iota(jnp.int32, (CH, CH), 0)
           >= lax.broadcasted_iota(jnp.int32, (CH, CH), 1)).astype(jnp.float32)
    base = jnp.zeros((1, E), jnp.float32)
    chunks = []
    for k in range(T // CH):
        ohk = oh_e[k * CH:(k + 1) * CH, :]
        chunks.append(lax.dot_general(tri, ohk, (((1,), (0,)), ((), ())),
                                      preferred_element_type=jnp.float32) + base)
        base = base + jnp.sum(ohk, axis=0, keepdims=True)
    cnt = jnp.concatenate(chunks, axis=0)
    pos = jnp.sum(cnt * oh_e, axis=1, keepdims=True) - 1.0
    keep = pos < float(C)
    pos_i = pos.astype(jnp.int32)

    iota_c = lax.broadcasted_iota(jnp.int32, (T, C_PAD), 1)
    oh_c = ((iota_c == pos_i) & keep).astype(jnp.float32)
    ti = lax.broadcasted_iota(jnp.int32, (T, 1), 0)
    hi_f = lax.shift_right_logical(ti, 8).astype(jnp.float32)
    lo_f = jnp.bitwise_and(ti, 255).astype(jnp.float32)
    g_hi = lax.dot_general(oh_e * hi_f, oh_c, (((0,), (0,)), ((), ())),
                           preferred_element_type=jnp.float32)
    g_lo = lax.dot_general(oh_e * lo_f, oh_c, (((0,), (0,)), ((), ())),
                           preferred_element_type=jnp.float32)
    pmat = lax.dot_general(oh_e * prb, oh_c, (((0,), (0,)), ((), ())),
                           preferred_element_type=jnp.float32)
    slot_iota = (lax.broadcasted_iota(jnp.int32, (E, C_PAD), 0) * C_PAD
                 + lax.broadcasted_iota(jnp.int32, (E, C_PAD), 1))
    fallback = jnp.bitwise_and(slot_iota, T - 1).astype(jnp.float32)
    g = g_hi * 256.0 + g_lo
    gmat_ref[...] = jnp.where(pmat > 0.0, g, fallback)
    pmat_ref[...] = pmat
    s2t_ref[...] = jnp.where(pmat > 0.0, g, -1.0)


def _router(x, gw, gb2):
    return pl.pallas_call(
        _router_body,
        out_shape=[
            jax.ShapeDtypeStruct((E, C_PAD), jnp.float32),
            jax.ShapeDtypeStruct((E, C_PAD), jnp.float32),
            jax.ShapeDtypeStruct((E, C_PAD), jnp.float32),
        ],
    )(x, gw, gb2)


@functools.cache
def _make_sc_gather(n_rows, d):
    rpw = n_rows // NW
    assert n_rows % (8 * NW) == 0
    mesh = plsc.VectorSubcoreMesh(core_axis_name="c", subcore_axis_name="s",
                                  num_cores=NUM_SC, num_subcores=NUM_SUBCORES)

    @functools.partial(
        pl.kernel,
        out_type=jax.ShapeDtypeStruct((n_rows, d), jnp.float32),
        mesh=mesh,
        scratch_types=[
            pltpu.VMEM((rpw,), jnp.int32),
            pltpu.VMEM((rpw, d), jnp.float32),
            pltpu.SemaphoreType.DMA,
        ],
    )
    def gather(table_hbm, idx_hbm, out_hbm, idx_v, rows_v, sem):
        wid = lax.axis_index("s") * NUM_SC + lax.axis_index("c")
        base = wid * rpw
        pltpu.sync_copy(idx_hbm.at[pl.ds(base, rpw)], idx_v)
        pltpu.async_copy(table_hbm.at[idx_v], rows_v, sem).wait()
        pltpu.sync_copy(rows_v, out_hbm.at[pl.ds(base, rpw)])

    return gather


def _sc_gather(table, idx):
    return _make_sc_gather(idx.shape[0], table.shape[1])(table, idx)


def _ffn_body(s2t_ref, xg_ref, w1_ref, b1_ref, w2_ref, b2_ref, pm_ref,
              out_ref, acc_ref):
    e = pl.program_id(0)
    f = pl.program_id(1)

    @pl.when((e == 0) & (f == 0))
    def _():
        out_ref[...] = jnp.zeros((T, D), jnp.float32)

    xv = xg_ref[0]
    h1 = lax.dot_general(xv, w1_ref[0], (((1,), (1,)), ((), ())),
                         preferred_element_type=jnp.float32)
    h1 = jnp.maximum(h1 + b1_ref[0], 0.0)
    part = lax.dot_general(h1, w2_ref[0], (((1,), (1,)), ((), ())),
                           preferred_element_type=jnp.float32)

    @pl.when(f == 0)
    def _():
        acc_ref[...] = part

    @pl.when(f > 0)
    def _():
        acc_ref[...] = acc_ref[...] + part

    @pl.when(f == NF - 1)
    def _():
        res = (acc_ref[...] + b2_ref[0]) * pm_ref[0]
        for c in range(C):
            tok = s2t_ref[e * C_PAD + c]

            @pl.when(tok >= 0)
            def _():
                out_ref[pl.ds(tok, 1), :] = res[c:c + 1, :]


def _ffn(s2t, xg3, w1, b1r, w2, b2r, pm3):
    grid_spec = pltpu.PrefetchScalarGridSpec(
        num_scalar_prefetch=1,
        grid=(E, NF),
        in_specs=[
            pl.BlockSpec((1, C_PAD, D), lambda e, f, s: (e, 0, 0)),
            pl.BlockSpec((1, FT, D), lambda e, f, s: (e, f, 0)),
            pl.BlockSpec((1, 1, FT), lambda e, f, s: (e, 0, f)),
            pl.BlockSpec((1, D, FT), lambda e, f, s: (e, 0, f)),
            pl.BlockSpec((1, 1, D), lambda e, f, s: (e, 0, 0)),
            pl.BlockSpec((1, C_PAD, 1), lambda e, f, s: (e, 0, 0)),
        ],
        out_specs=pl.BlockSpec((T, D), lambda e, f, s: (0, 0)),
        scratch_shapes=[pltpu.VMEM((C_PAD, D), jnp.float32)],
    )
    return pl.pallas_call(
        _ffn_body,
        grid_spec=grid_spec,
        out_shape=jax.ShapeDtypeStruct((T, D), jnp.float32),
        compiler_params=pltpu.CompilerParams(
            dimension_semantics=("arbitrary", "arbitrary")),
    )(s2t, xg3, w1, b1r, w2, b2r, pm3)


def kernel(h, gate_w, gate_b, w1, b1, w2, b2):
    x = h.reshape(T, D)
    gmat, pmat, s2t = _router(x, gate_w, gate_b.reshape(1, E))
    gidx = gmat.reshape(S).astype(jnp.int32)
    xg = _sc_gather(x, gidx)
    out = _ffn(s2t.reshape(S).astype(jnp.int32), xg.reshape(E, C_PAD, D),
               w1, b1.reshape(E, 1, F), w2, b2.reshape(E, 1, D),
               pmat.reshape(E, C_PAD, 1))
    return out.reshape(B, L, D)
